# baseline (speedup 1.0000x reference)
import functools

import jax
import jax.numpy as jnp
from jax import lax
from jax.experimental import pallas as pl
from jax.experimental.pallas import tpu as pltpu
from jax.experimental.pallas import tpu_sc as plsc

N = 10000
E = 320000
D = 128

NC = 2
NS = 16
C = 80
NCHUNK = E // C
SHARE = 640
ZC = 16


def _leaky(x):
    return jnp.where(x >= 0, x, 0.2 * x)


def _scores_body(feat_ref, aw_ref, out_ref):
    out_ref[...] = jnp.dot(feat_ref[...], aw_ref[...],
                           preferred_element_type=jnp.float32)


def _node_scores(feat_all, aw2):
    br = 2000
    return pl.pallas_call(
        _scores_body,
        grid=(2 * N // br,),
        in_specs=[
            pl.BlockSpec((br, D), lambda r: (r, 0)),
            pl.BlockSpec((D, 2), lambda r: (0, 0)),
        ],
        out_specs=pl.BlockSpec((br, 2), lambda r: (r, 0)),
        out_shape=jax.ShapeDtypeStruct((2 * N, 2), jnp.float32),
    )(feat_all, aw2)


def _edge_body(feat_hbm, scoreA_hbm, scoreB_hbm, src_hbm, dst_hbm, norm_hbm,
               s2_out, s_out, s0_out, *scr):
    scoreA_v, scoreB_v, zrows_v, zbuf_v = scr[:4]
    srcg_b, dst_b, norm_b, gidx_b, didx_b, e_b, v_b, rows_b = (
        scr[4 + 2 * t: 6 + 2 * t] for t in range(8))
    s2_sh, s_sh, s0_sh = scr[20:23]
    sem_lin = scr[23:25]
    sem_g = scr[25:27]
    sem_sc = scr[27:29]
    cid = lax.axis_index("c")
    sid = lax.axis_index("s")

    def zrow(i, _):
        for j in range(D // 16):
            zrows_v[i, pl.ds(j * 16, 16)] = jnp.zeros((16,), jnp.float32)
        return 0
    lax.fori_loop(0, ZC, zrow, 0)
    for i in range(ZC // 16):
        zbuf_v[pl.ds(i * 16, 16)] = jnp.zeros((16,), jnp.float32)

    share_lo = sid * SHARE
    for k in range(SHARE // ZC):
        off = share_lo + k * ZC
        @pl.when(off < N)
        def _():
            pltpu.sync_copy(zrows_v, s2_sh.at[pl.ds(off, ZC)])
            pltpu.sync_copy(zbuf_v, s_sh.at[pl.ds(off, ZC)])
            pltpu.sync_copy(zbuf_v, s0_sh.at[pl.ds(off, ZC)])

    pltpu.sync_copy(scoreA_hbm.at[pl.ds(cid * N, N)], scoreA_v)
    pltpu.sync_copy(scoreB_hbm.at[pl.ds((1 - cid) * N, N)], scoreB_v)
    plsc.subcore_barrier()

    foff = jnp.broadcast_to(cid * N, (16,)).astype(jnp.int32)
    nj = jnp.where(sid < NCHUNK % NS, NCHUNK // NS + 1, NCHUNK // NS)

    def issue_loads(j, p):
        eb = cid * E + (sid + j * NS) * C
        pltpu.async_copy(src_hbm.at[pl.ds(eb, C)], srcg_b[p], sem_lin[p])
        pltpu.async_copy(dst_hbm.at[pl.ds(eb, C)], dst_b[p], sem_lin[p])
        pltpu.async_copy(norm_hbm.at[pl.ds(eb, C)], norm_b[p], sem_lin[p])

    def drain_scatters(p):
        pltpu.make_async_copy(rows_b[p], s2_sh.at[didx_b[p]], sem_sc[p]).wait()
        pltpu.make_async_copy(e_b[p], s_sh.at[didx_b[p]], sem_sc[p]).wait()
        pltpu.make_async_copy(v_b[p], s0_sh.at[didx_b[p]], sem_sc[p]).wait()

    def step(j, p):
        q = 1 - p
        @pl.when(j + 1 < nj)
        def _():
            issue_loads(j + 1, q)
        pltpu.make_async_copy(src_hbm.at[pl.ds(0, C)], srcg_b[p],
                              sem_lin[p]).wait()
        pltpu.make_async_copy(src_hbm.at[pl.ds(0, C)], dst_b[p],
                              sem_lin[p]).wait()
        pltpu.make_async_copy(norm_hbm.at[pl.ds(0, C)], norm_b[p],
                              sem_lin[p]).wait()
        @pl.when(j >= 2)
        def _():
            drain_scatters(p)
        for i in range(C // 16):
            sl = pl.ds(i * 16, 16)
            gidx_b[p][sl] = srcg_b[p][sl] + foff
            didx_b[p][sl] = dst_b[p][sl]
        gth = pltpu.async_copy(feat_hbm.at[gidx_b[p]], rows_b[p], sem_g[p])
        for i in range(C // 16):
            sl = pl.ds(i * 16, 16)
            sa = plsc.load_gather(scoreA_v, [srcg_b[p][sl]])
            sb = plsc.load_gather(scoreB_v, [dst_b[p][sl]])
            e = jnp.exp(_leaky(sa + sb))
            e_b[p][sl] = e
            v_b[p][sl] = e * norm_b[p][sl]
        gth.wait()

        @plsc.parallel_loop(0, C, 1, unroll=4)
        def srow(i):
            idx = jnp.broadcast_to(i, (16,)).astype(jnp.int32)
            w = plsc.load_gather(v_b[p], [idx])
            for jj in range(D // 16):
                sl = pl.ds(jj * 16, 16)
                rows_b[p][i, sl] = rows_b[p][i, sl] * w

        pltpu.async_copy(rows_b[p], s2_sh.at[didx_b[p]], sem_sc[p], add=True)
        pltpu.async_copy(e_b[p], s_sh.at[didx_b[p]], sem_sc[p], add=True)
        pltpu.async_copy(v_b[p], s0_sh.at[didx_b[p]], sem_sc[p], add=True)

    issue_loads(0, 0)

    def body2(k, _):
        for c in range(2):
            j = 2 * k + c
            @pl.when(j < nj)
            def _():
                step(j, c)
        return 0
    lax.fori_loop(0, (NCHUNK // NS + 2) // 2, body2, 0)

    for p in range(2):
        drain_scatters(p)

    plsc.subcore_barrier()
    for k in range(SHARE // ZC):
        off = share_lo + k * ZC
        @pl.when(off < N)
        def _():
            pltpu.sync_copy(s2_sh.at[pl.ds(off, ZC)], zrows_v)
            pltpu.sync_copy(zrows_v, s2_out.at[pl.ds(cid * N + off, ZC)])
            pltpu.sync_copy(s_sh.at[pl.ds(off, ZC)], zbuf_v)
            pltpu.sync_copy(zbuf_v, s_out.at[pl.ds(cid * N + off, ZC)])
            pltpu.sync_copy(s0_sh.at[pl.ds(off, ZC)], zbuf_v)
            pltpu.sync_copy(zbuf_v, s0_out.at[pl.ds(cid * N + off, ZC)])


def _edge_pass(feat_all, scoreA, scoreB, src_all, dst_all, norm_all):
    mesh = plsc.VectorSubcoreMesh(core_axis_name="c", subcore_axis_name="s")
    f32 = jnp.float32
    run = pl.kernel(
        _edge_body,
        out_type=(
            jax.ShapeDtypeStruct((2 * N, D), f32),
            jax.ShapeDtypeStruct((2 * N,), f32),
            jax.ShapeDtypeStruct((2 * N,), f32),
        ),
        mesh=mesh,
        compiler_params=pltpu.CompilerParams(needs_layout_passes=False),
        scratch_types=(
            [
                pltpu.VMEM((N,), f32),
                pltpu.VMEM((N,), f32),
                pltpu.VMEM((ZC, D), f32),
                pltpu.VMEM((ZC,), f32),
            ]
            + [pltpu.VMEM((C,), jnp.int32)] * 2
            + [pltpu.VMEM((C,), jnp.int32)] * 2
            + [pltpu.VMEM((C,), f32)] * 2
            + [pltpu.VMEM((C,), jnp.int32)] * 2
            + [pltpu.VMEM((C,), jnp.int32)] * 2
            + [pltpu.VMEM((C,), f32)] * 2
            + [pltpu.VMEM((C,), f32)] * 2
            + [pltpu.VMEM((C, D), f32)] * 2
            + [
                pltpu.VMEM_SHARED((N, D), f32),
                pltpu.VMEM_SHARED((N,), f32),
                pltpu.VMEM_SHARED((N,), f32),
            ]
            + [pltpu.SemaphoreType.DMA] * 6
        ),
    )
    return run(feat_all, scoreA, scoreB, src_all, dst_all, norm_all)


def _post_body(s2_ref, fd_ref, s_ref, s0_ref, w1_ref, w2_ref, bsum_ref,
               out_ref):
    x = s2_ref[0]
    num = jnp.dot(x, w1_ref[...], preferred_element_type=jnp.float32)
    num += jnp.dot(x * fd_ref[0], w2_ref[...],
                   preferred_element_type=jnp.float32)
    num += s0_ref[0] * bsum_ref[...]
    h = _leaky(num / (s_ref[0] + 1e-16))
    nrm = jnp.sqrt(jnp.sum(h * h, axis=1, keepdims=True))
    out_ref[0] = h / jnp.maximum(nrm, 1e-12)


def _post(s2, feat_all, s, s0, w1, w2, bsum):
    br = 2000
    return pl.pallas_call(
        _post_body,
        grid=(2, N // br),
        in_specs=[
            pl.BlockSpec((1, br, D), lambda c, r: (c, r, 0)),
            pl.BlockSpec((1, br, D), lambda c, r: (1 - c, r, 0)),
            pl.BlockSpec((1, br, 1), lambda c, r: (c, r, 0)),
            pl.BlockSpec((1, br, 1), lambda c, r: (c, r, 0)),
            pl.BlockSpec((D, D), lambda c, r: (0, 0)),
            pl.BlockSpec((D, D), lambda c, r: (0, 0)),
            pl.BlockSpec((1, D), lambda c, r: (0, 0)),
        ],
        out_specs=pl.BlockSpec((1, br, D), lambda c, r: (c, r, 0)),
        out_shape=jax.ShapeDtypeStruct((2, N, D), jnp.float32),
    )(s2, feat_all, s, s0, w1, w2, bsum)


@jax.jit
def kernel(feat_user, feat_item, edge_src_ui, edge_dst_ui, norm_ui, norm_iu,
           W1_w, W1_b, W2_w, W2_b, attn_w):
    feat_all = jnp.concatenate([feat_user, feat_item], axis=0)
    aw2 = jnp.concatenate([attn_w[:D], attn_w[D:]], axis=1)
    scores = _node_scores(feat_all, aw2)
    scoreA = scores[:, 0]
    scoreB = scores[:, 1]

    src_all = jnp.concatenate([edge_src_ui, edge_dst_ui]).astype(jnp.int32)
    dst_all = jnp.concatenate([edge_dst_ui, edge_src_ui]).astype(jnp.int32)
    norm_all = jnp.concatenate([norm_ui[:, 0], norm_iu[:, 0]])

    s2, s, s0 = _edge_pass(feat_all, scoreA, scoreB, src_all, dst_all,
                           norm_all)

    bsum = (W1_b + W2_b).reshape(1, D)
    out = _post(s2.reshape(2, N, D), feat_all.reshape(2, N, D),
                s.reshape(2, N, 1), s0.reshape(2, N, 1), W1_w, W2_w, bsum)
    return out[1], out[0]


# speedup vs baseline: 1.0124x; 1.0124x over previous
import functools

import jax
import jax.numpy as jnp
from jax import lax
from jax.experimental import pallas as pl
from jax.experimental.pallas import tpu as pltpu
from jax.experimental.pallas import tpu_sc as plsc

N = 10000
E = 320000
D = 128

NC = 2
NS = 16
C = 80
NCHUNK = E // C
SHARE = 640
ZC = 16


def _leaky(x):
    return jnp.where(x >= 0, x, 0.2 * x)


def _scores_body(feat_ref, aw_ref, out_ref):
    out_ref[...] = jnp.dot(feat_ref[...], aw_ref[...],
                           preferred_element_type=jnp.float32)


def _node_scores(feat_all, aw2):
    br = 2000
    return pl.pallas_call(
        _scores_body,
        grid=(2 * N // br,),
        in_specs=[
            pl.BlockSpec((br, D), lambda r: (r, 0)),
            pl.BlockSpec((D, 2), lambda r: (0, 0)),
        ],
        out_specs=pl.BlockSpec((br, 2), lambda r: (r, 0)),
        out_shape=jax.ShapeDtypeStruct((2 * N, 2), jnp.float32),
    )(feat_all, aw2)


def _edge_body(feat_hbm, scoreA_hbm, scoreB_hbm, src_hbm, dst_hbm, norm_hbm,
               s2_out, s_out, s0_out, *scr):
    scoreA_v, scoreB_v, zrows_v, zbuf_v = scr[:4]
    srcg_b, dst_b, norm_b, gidx_b, didx_b, e_b, v_b, rows_b = (
        scr[4 + 2 * t: 6 + 2 * t] for t in range(8))
    s2_sh, s_sh, s0_sh = scr[20:23]
    sem_lin = scr[23:25]
    sem_g = scr[25:27]
    sem_sc = scr[27:29]
    cid = lax.axis_index("c")
    sid = lax.axis_index("s")

    def zrow(i, _):
        for j in range(D // 16):
            zrows_v[i, pl.ds(j * 16, 16)] = jnp.zeros((16,), jnp.float32)
        return 0
    lax.fori_loop(0, ZC, zrow, 0)
    for i in range(ZC // 16):
        zbuf_v[pl.ds(i * 16, 16)] = jnp.zeros((16,), jnp.float32)

    share_lo = sid * SHARE
    for k in range(SHARE // ZC):
        off = share_lo + k * ZC
        @pl.when(off < N)
        def _():
            pltpu.sync_copy(zrows_v, s2_sh.at[pl.ds(off, ZC)])
            pltpu.sync_copy(zbuf_v, s_sh.at[pl.ds(off, ZC)])
            pltpu.sync_copy(zbuf_v, s0_sh.at[pl.ds(off, ZC)])

    pltpu.sync_copy(scoreA_hbm.at[pl.ds(cid * N, N)], scoreA_v)
    pltpu.sync_copy(scoreB_hbm.at[pl.ds((1 - cid) * N, N)], scoreB_v)
    plsc.subcore_barrier()

    foff = jnp.broadcast_to(cid * N, (16,)).astype(jnp.int32)
    nj = jnp.where(sid < NCHUNK % NS, NCHUNK // NS + 1, NCHUNK // NS)

    def issue_loads(j, p):
        eb = cid * E + (sid + j * NS) * C
        pltpu.async_copy(src_hbm.at[pl.ds(eb, C)], srcg_b[p], sem_lin[p])
        pltpu.async_copy(dst_hbm.at[pl.ds(eb, C)], dst_b[p], sem_lin[p])
        pltpu.async_copy(norm_hbm.at[pl.ds(eb, C)], norm_b[p], sem_lin[p])

    def drain_scatters(p):
        pltpu.make_async_copy(rows_b[p], s2_sh.at[didx_b[p]], sem_sc[p]).wait()
        pltpu.make_async_copy(e_b[p], s_sh.at[didx_b[p]], sem_sc[p]).wait()
        pltpu.make_async_copy(v_b[p], s0_sh.at[didx_b[p]], sem_sc[p]).wait()

    def step(j, p):
        q = 1 - p
        @pl.when(j + 1 < nj)
        def _():
            issue_loads(j + 1, q)
        pltpu.make_async_copy(src_hbm.at[pl.ds(0, C)], srcg_b[p],
                              sem_lin[p]).wait()
        pltpu.make_async_copy(src_hbm.at[pl.ds(0, C)], dst_b[p],
                              sem_lin[p]).wait()
        pltpu.make_async_copy(norm_hbm.at[pl.ds(0, C)], norm_b[p],
                              sem_lin[p]).wait()
        @pl.when(j >= 2)
        def _():
            drain_scatters(p)
        for i in range(C // 16):
            sl = pl.ds(i * 16, 16)
            gidx_b[p][sl] = srcg_b[p][sl] + foff
            didx_b[p][sl] = dst_b[p][sl]
        gth = pltpu.async_copy(feat_hbm.at[gidx_b[p]], rows_b[p], sem_g[p])
        for i in range(C // 16):
            sl = pl.ds(i * 16, 16)
            sa = plsc.load_gather(scoreA_v, [srcg_b[p][sl]])
            sb = plsc.load_gather(scoreB_v, [dst_b[p][sl]])
            e = jnp.exp(_leaky(sa + sb))
            e_b[p][sl] = e
            v_b[p][sl] = e * norm_b[p][sl]
        gth.wait()

        @plsc.parallel_loop(0, C, 1, unroll=4)
        def srow(i):
            idx = jnp.broadcast_to(i, (16,)).astype(jnp.int32)
            w = plsc.load_gather(v_b[p], [idx])
            for jj in range(D // 16):
                sl = pl.ds(jj * 16, 16)
                rows_b[p][i, sl] = rows_b[p][i, sl] * w

        pltpu.async_copy(rows_b[p], s2_sh.at[didx_b[p]], sem_sc[p], add=True)
        pltpu.async_copy(e_b[p], s_sh.at[didx_b[p]], sem_sc[p], add=True)
        pltpu.async_copy(v_b[p], s0_sh.at[didx_b[p]], sem_sc[p], add=True)

    issue_loads(0, 0)

    def body2(k, _):
        for c in range(2):
            j = 2 * k + c
            @pl.when(j < nj)
            def _():
                step(j, c)
        return 0
    lax.fori_loop(0, (NCHUNK // NS + 2) // 2, body2, 0)

    for p in range(2):
        drain_scatters(p)

    plsc.subcore_barrier()
    for k in range(SHARE // ZC):
        off = share_lo + k * ZC
        @pl.when(off < N)
        def _():
            pltpu.sync_copy(s2_sh.at[pl.ds(off, ZC)], zrows_v)
            pltpu.sync_copy(zrows_v, s2_out.at[pl.ds(cid * N + off, ZC)])
            pltpu.sync_copy(s_sh.at[pl.ds(off, ZC)], zbuf_v)
            pltpu.sync_copy(zbuf_v, s_out.at[pl.ds(cid * N + off, ZC)])
            pltpu.sync_copy(s0_sh.at[pl.ds(off, ZC)], zbuf_v)
            pltpu.sync_copy(zbuf_v, s0_out.at[pl.ds(cid * N + off, ZC)])


def _edge_pass(feat_all, scoreA, scoreB, src_all, dst_all, norm_all):
    mesh = plsc.VectorSubcoreMesh(core_axis_name="c", subcore_axis_name="s")
    f32 = jnp.float32
    run = pl.kernel(
        _edge_body,
        out_type=(
            jax.ShapeDtypeStruct((2 * N, D), f32),
            jax.ShapeDtypeStruct((2 * N,), f32),
            jax.ShapeDtypeStruct((2 * N,), f32),
        ),
        mesh=mesh,
        compiler_params=pltpu.CompilerParams(needs_layout_passes=False),
        scratch_types=(
            [
                pltpu.VMEM((N,), f32),
                pltpu.VMEM((N,), f32),
                pltpu.VMEM((ZC, D), f32),
                pltpu.VMEM((ZC,), f32),
            ]
            + [pltpu.VMEM((C,), jnp.int32)] * 2
            + [pltpu.VMEM((C,), jnp.int32)] * 2
            + [pltpu.VMEM((C,), f32)] * 2
            + [pltpu.VMEM((C,), jnp.int32)] * 2
            + [pltpu.VMEM((C,), jnp.int32)] * 2
            + [pltpu.VMEM((C,), f32)] * 2
            + [pltpu.VMEM((C,), f32)] * 2
            + [pltpu.VMEM((C, D), f32)] * 2
            + [
                pltpu.VMEM_SHARED((N, D), f32),
                pltpu.VMEM_SHARED((N,), f32),
                pltpu.VMEM_SHARED((N,), f32),
            ]
            + [pltpu.SemaphoreType.DMA] * 6
        ),
    )
    return run(feat_all, scoreA, scoreB, src_all, dst_all, norm_all)


def _post_body(s2_ref, fd_ref, s_ref, s0_ref, w1_ref, w2_ref, bsum_ref,
               out_ref):
    x = s2_ref[...]
    num = jnp.dot(x, w1_ref[...], preferred_element_type=jnp.float32)
    num += jnp.dot(x * fd_ref[...], w2_ref[...],
                   preferred_element_type=jnp.float32)
    num += s0_ref[...] * bsum_ref[...]
    h = _leaky(num / (s_ref[...] + 1e-16))
    nrm = jnp.sqrt(jnp.sum(h * h, axis=1, keepdims=True))
    out_ref[...] = h / jnp.maximum(nrm, 1e-12)


def _post(s2, feat_dst, s, s0, w1, w2, bsum):
    br = 2000
    return pl.pallas_call(
        _post_body,
        grid=(2 * N // br,),
        in_specs=[
            pl.BlockSpec((br, D), lambda r: (r, 0)),
            pl.BlockSpec((br, D), lambda r: (r, 0)),
            pl.BlockSpec((br, 1), lambda r: (r, 0)),
            pl.BlockSpec((br, 1), lambda r: (r, 0)),
            pl.BlockSpec((D, D), lambda r: (0, 0)),
            pl.BlockSpec((D, D), lambda r: (0, 0)),
            pl.BlockSpec((1, D), lambda r: (0, 0)),
        ],
        out_specs=pl.BlockSpec((br, D), lambda r: (r, 0)),
        out_shape=jax.ShapeDtypeStruct((2 * N, D), jnp.float32),
    )(s2, feat_dst, s, s0, w1, w2, bsum)


@jax.jit
def kernel(feat_user, feat_item, edge_src_ui, edge_dst_ui, norm_ui, norm_iu,
           W1_w, W1_b, W2_w, W2_b, attn_w):
    feat_all = jnp.concatenate([feat_user, feat_item], axis=0)
    aw2 = jnp.concatenate([attn_w[:D], attn_w[D:]], axis=1)
    scores = _node_scores(feat_all, aw2)
    scoreA = scores[:, 0]
    scoreB = scores[:, 1]

    src_all = jnp.concatenate([edge_src_ui, edge_dst_ui]).astype(jnp.int32)
    dst_all = jnp.concatenate([edge_dst_ui, edge_src_ui]).astype(jnp.int32)
    norm_all = jnp.concatenate([norm_ui[:, 0], norm_iu[:, 0]])

    s2, s, s0 = _edge_pass(feat_all, scoreA, scoreB, src_all, dst_all,
                           norm_all)

    feat_dst = jnp.concatenate([feat_item, feat_user], axis=0)
    bsum = (W1_b + W2_b).reshape(1, D)
    out = _post(s2, feat_dst, s.reshape(2 * N, 1), s0.reshape(2 * N, 1),
                W1_w, W2_w, bsum)
    return out[N:], out[:N]
